# Initial kernel scaffold; baseline (speedup 1.0000x reference)
import jax
import jax.numpy as jnp
from jax.experimental import pallas as pl


def kernel(positions, volumes, distances, radialDistances, density, i, j):
    raise NotImplementedError("write your pallas kernel here")


# speedup vs baseline: 156.9942x; 156.9942x over previous
import functools

import numpy as np
import jax
import jax.numpy as jnp
from jax import lax
from jax.experimental import pallas as pl
from jax.experimental.pallas import tpu as pltpu
from jax.experimental.pallas import tpu_sc as plsc

N = 100000
E = 3200000
SUPPORT = 0.025
DELTA = 0.1
C0 = float(10.0 * np.sqrt(2.0 * 9.81 * 0.3))
EPS = float(SUPPORT ** 2 * 0.1)
REST_DENSITY = 1000.0
WEND_S = float((7.0 / np.pi) / SUPPORT ** 3)
OUT_SCALE = float(SUPPORT * DELTA * C0)

LANES = 16
NC = 2
NS = 16
NW = NC * NS
EW = E // NW
B = 2000
CHUNKS = EW // B
NP_PAD = 100352
PER_TILE = NP_PAD // NS
ROWS = NP_PAD // 128
GRID = ROWS // 8


def _mesh():
    return plsc.VectorSubcoreMesh(
        core_axis_name="c", subcore_axis_name="s", num_cores=NC, num_subcores=NS
    )


def _zero_accs(zbuf, accs, tile_base):
    @pl.loop(0, PER_TILE // LANES)
    def _(k):
        zbuf[pl.ds(k * LANES, LANES)] = jnp.zeros((LANES,), jnp.float32)

    for acc in accs:
        pltpu.sync_copy(zbuf, acc.at[pl.ds(tile_base, PER_TILE)])


def _dump_accs(zbuf, accs, outs, tile_base, out_base):
    for acc, oh in zip(accs, outs):
        pltpu.sync_copy(acc.at[pl.ds(tile_base, PER_TILE)], zbuf)
        pltpu.sync_copy(zbuf, oh.at[pl.ds(out_base, PER_TILE)])


def _wait_all(descs):
    for d in descs:
        d.wait()


def _sweep_norm(ii_h, jj_h, dx_h, dy_h, rad_h, px_h, py_h, vol_h,
                o00_h, o01_h, o10_h, o11_h,
                ii, jj, dx, dy, rad, xi, yi, xj, yj, vj,
                c00, c01, c10, c11, zbuf, a00, a01, a10, a11, sem):
    c = lax.axis_index("c")
    s = lax.axis_index("s")
    wid = s * NC + c
    tile_base = s * PER_TILE
    accs = (a00, a01, a10, a11)
    _zero_accs(zbuf, accs, tile_base)
    plsc.subcore_barrier()

    base = wid * EW

    @pl.loop(0, CHUNKS)
    def _(ch):
        sl = pl.ds(base + ch * B, B)
        _wait_all([pltpu.async_copy(h.at[sl], v, sem)
                   for h, v in ((ii_h, ii), (jj_h, jj), (dx_h, dx),
                                (dy_h, dy), (rad_h, rad))])
        _wait_all([pltpu.async_copy(px_h.at[ii], xi, sem),
                   pltpu.async_copy(py_h.at[ii], yi, sem),
                   pltpu.async_copy(px_h.at[jj], xj, sem),
                   pltpu.async_copy(py_h.at[jj], yj, sem),
                   pltpu.async_copy(vol_h.at[jj], vj, sem)])

        @pl.loop(0, B // LANES)
        def _(k):
            v16 = pl.ds(k * LANES, LANES)
            q = rad[v16]
            t = 1.0 - q
            gwc = WEND_S * ((-20.0 * q) * ((t * t) * t))
            gx = gwc * dx[v16]
            gy = gwc * dy[v16]
            rx = xj[v16] - xi[v16]
            ry = yj[v16] - yi[v16]
            v = vj[v16]
            c00[v16] = (rx * gx) * v
            c01[v16] = (rx * gy) * v
            c10[v16] = (ry * gx) * v
            c11[v16] = (ry * gy) * v

        pltpu.sync_copy(c00, a00.at[ii], add=True)
        pltpu.sync_copy(c01, a01.at[ii], add=True)
        pltpu.sync_copy(c10, a10.at[ii], add=True)
        pltpu.sync_copy(c11, a11.at[ii], add=True)

    plsc.subcore_barrier()
    _dump_accs(zbuf, accs, (o00_h, o01_h, o10_h, o11_h),
               tile_base, c * NP_PAD + tile_base)


def _sweep_grad(ii_h, jj_h, dx_h, dy_h, rad_h, l00_h, l01_h, l10_h, l11_h,
                den_h, vol_h, ox_h, oy_h,
                ii, jj, dx, dy, rad, l00, l01, l10, l11, di, dj, vj,
                cx, cy, zbuf, ax, ay, sem):
    c = lax.axis_index("c")
    s = lax.axis_index("s")
    wid = s * NC + c
    tile_base = s * PER_TILE
    accs = (ax, ay)
    _zero_accs(zbuf, accs, tile_base)
    plsc.subcore_barrier()

    base = wid * EW

    @pl.loop(0, CHUNKS)
    def _(ch):
        sl = pl.ds(base + ch * B, B)
        _wait_all([pltpu.async_copy(h.at[sl], v, sem)
                   for h, v in ((ii_h, ii), (jj_h, jj), (dx_h, dx),
                                (dy_h, dy), (rad_h, rad))])
        _wait_all([pltpu.async_copy(l00_h.at[ii], l00, sem),
                   pltpu.async_copy(l01_h.at[ii], l01, sem),
                   pltpu.async_copy(l10_h.at[ii], l10, sem),
                   pltpu.async_copy(l11_h.at[ii], l11, sem),
                   pltpu.async_copy(den_h.at[ii], di, sem),
                   pltpu.async_copy(den_h.at[jj], dj, sem),
                   pltpu.async_copy(vol_h.at[jj], vj, sem)])

        @pl.loop(0, B // LANES)
        def _(k):
            v16 = pl.ds(k * LANES, LANES)
            q = rad[v16]
            t = 1.0 - q
            gwc = WEND_S * ((-20.0 * q) * ((t * t) * t))
            gx = gwc * dx[v16]
            gy = gwc * dy[v16]
            drho = dj[v16] * REST_DENSITY - di[v16] * REST_DENSITY
            coef = (drho * vj[v16]) * 2.0
            cx[v16] = -(coef * (l00[v16] * gx + l01[v16] * gy))
            cy[v16] = -(coef * (l10[v16] * gx + l11[v16] * gy))

        pltpu.sync_copy(cx, ax.at[ii], add=True)
        pltpu.sync_copy(cy, ay.at[ii], add=True)

    plsc.subcore_barrier()
    _dump_accs(zbuf, accs, (ox_h, oy_h), tile_base, c * NP_PAD + tile_base)


def _sweep_diff(ii_h, jj_h, dx_h, dy_h, rad_h, px_h, py_h, den_h, vol_h,
                rgx_h, rgy_h, od_h,
                ii, jj, dx, dy, rad, xi, yi, xj, yj, di, dj, vj,
                rgxi, rgyi, rgxj, rgyj, cd, zbuf, ad, sem):
    c = lax.axis_index("c")
    s = lax.axis_index("s")
    wid = s * NC + c
    tile_base = s * PER_TILE
    _zero_accs(zbuf, (ad,), tile_base)
    plsc.subcore_barrier()

    base = wid * EW

    @pl.loop(0, CHUNKS)
    def _(ch):
        sl = pl.ds(base + ch * B, B)
        _wait_all([pltpu.async_copy(h.at[sl], v, sem)
                   for h, v in ((ii_h, ii), (jj_h, jj), (dx_h, dx),
                                (dy_h, dy), (rad_h, rad))])
        _wait_all([pltpu.async_copy(px_h.at[ii], xi, sem),
                   pltpu.async_copy(py_h.at[ii], yi, sem),
                   pltpu.async_copy(px_h.at[jj], xj, sem),
                   pltpu.async_copy(py_h.at[jj], yj, sem),
                   pltpu.async_copy(den_h.at[ii], di, sem),
                   pltpu.async_copy(den_h.at[jj], dj, sem),
                   pltpu.async_copy(vol_h.at[jj], vj, sem),
                   pltpu.async_copy(rgx_h.at[ii], rgxi, sem),
                   pltpu.async_copy(rgy_h.at[ii], rgyi, sem),
                   pltpu.async_copy(rgx_h.at[jj], rgxj, sem),
                   pltpu.async_copy(rgy_h.at[jj], rgyj, sem)])

        @pl.loop(0, B // LANES)
        def _(k):
            v16 = pl.ds(k * LANES, LANES)
            q = rad[v16]
            t = 1.0 - q
            gwc = WEND_S * ((-20.0 * q) * ((t * t) * t))
            gx = gwc * dx[v16]
            gy = gwc * dy[v16]
            rx = xj[v16] - xi[v16]
            ry = yj[v16] - yi[v16]
            r2 = (rx * rx + ry * ry) + EPS
            drho = dj[v16] * REST_DENSITY - di[v16] * REST_DENSITY
            sc = (2.0 * drho) / r2
            psx = sc * rx - (rgxi[v16] + rgxj[v16])
            psy = sc * ry - (rgyi[v16] + rgyj[v16])
            cd[v16] = (psx * gx + psy * gy) * vj[v16]

        pltpu.sync_copy(cd, ad.at[ii], add=True)

    plsc.subcore_barrier()
    _dump_accs(zbuf, (ad,), (od_h,), tile_base, c * NP_PAD + tile_base)


def _pinv_body(n00a, n00b, n01a, n01b, n10a, n10b, n11a, n11b,
               l00, l01, l10, l11):
    m00 = n00a[...] + n00b[...]
    m01 = n01a[...] + n01b[...]
    m10 = n10a[...] + n10b[...]
    m11 = n11a[...] + n11b[...]
    det = m00 * m11 - m01 * m10
    dets = jnp.where(jnp.abs(det) > 1e-7, det,
                     jnp.where(det >= 0, 1e-7, -1e-7))
    l00[...] = m11 / dets
    l01[...] = -m01 / dets
    l10[...] = -m10 / dets
    l11[...] = m00 / dets


def _addscale_body(pa, pb, o, *, scale):
    o[...] = (pa[...] + pb[...]) * scale


def _tc_pinv(parts):
    bs = pl.BlockSpec((8, 128), lambda g: (g, 0))
    f = pl.pallas_call(
        _pinv_body,
        grid=(GRID,),
        in_specs=[bs] * 8,
        out_specs=[bs] * 4,
        out_shape=[jax.ShapeDtypeStruct((ROWS, 128), jnp.float32)] * 4,
    )
    return f(*parts)


def _tc_addscale(pa, pb, scale):
    bs = pl.BlockSpec((8, 128), lambda g: (g, 0))
    f = pl.pallas_call(
        functools.partial(_addscale_body, scale=scale),
        grid=(GRID,),
        in_specs=[bs, bs],
        out_specs=bs,
        out_shape=jax.ShapeDtypeStruct((ROWS, 128), jnp.float32),
    )
    return f(pa, pb)


def _split2(flat):
    r = flat.reshape(2, ROWS, 128)
    return r[0], r[1]


def kernel(positions, volumes, distances, radialDistances, density, i, j):
    ii = i.astype(jnp.int32)
    jj = j.astype(jnp.int32)
    pad = NP_PAD - N
    px = jnp.pad(positions[:, 0], (0, pad))
    py = jnp.pad(positions[:, 1], (0, pad))
    vol = jnp.pad(volumes, (0, pad))
    den = jnp.pad(density, (0, pad))
    dx = distances[:, 0]
    dy = distances[:, 1]
    rad = radialDistances

    mesh = _mesh()
    ebuf = lambda: pltpu.VMEM((B,), jnp.float32)
    eibuf = lambda: pltpu.VMEM((B,), jnp.int32)
    zbuf = pltpu.VMEM((PER_TILE,), jnp.float32)
    acc = lambda: pltpu.VMEM_SHARED((NP_PAD,), jnp.float32)
    oshape = jax.ShapeDtypeStruct((2 * NP_PAD,), jnp.float32)

    sweep1 = pl.kernel(
        _sweep_norm,
        out_type=(oshape,) * 4,
        mesh=mesh,
        scratch_types=(
            [eibuf(), eibuf()] + [ebuf()] * 12
            + [zbuf, acc(), acc(), acc(), acc(), pltpu.SemaphoreType.DMA]
        ),
    )
    nm00, nm01, nm10, nm11 = sweep1(ii, jj, dx, dy, rad, px, py, vol)

    parts = (_split2(nm00) + _split2(nm01) + _split2(nm10) + _split2(nm11))
    l00, l01, l10, l11 = _tc_pinv(parts)
    l00 = l00.reshape(NP_PAD)
    l01 = l01.reshape(NP_PAD)
    l10 = l10.reshape(NP_PAD)
    l11 = l11.reshape(NP_PAD)

    sweep2 = pl.kernel(
        _sweep_grad,
        out_type=(oshape,) * 2,
        mesh=mesh,
        scratch_types=(
            [eibuf(), eibuf()] + [ebuf()] * 12
            + [zbuf, acc(), acc(), pltpu.SemaphoreType.DMA]
        ),
    )
    rgx2, rgy2 = sweep2(ii, jj, dx, dy, rad, l00, l01, l10, l11, den, vol)
    rgx = _tc_addscale(*_split2(rgx2), 1.0).reshape(NP_PAD)
    rgy = _tc_addscale(*_split2(rgy2), 1.0).reshape(NP_PAD)

    sweep3 = pl.kernel(
        _sweep_diff,
        out_type=oshape,
        mesh=mesh,
        scratch_types=(
            [eibuf(), eibuf()] + [ebuf()] * 15
            + [zbuf, acc(), pltpu.SemaphoreType.DMA]
        ),
    )
    dd2 = sweep3(ii, jj, dx, dy, rad, px, py, den, vol, rgx, rgy)
    out = _tc_addscale(*_split2(dd2), OUT_SCALE).reshape(NP_PAD)
    return out[:N]


# speedup vs baseline: 214.4703x; 1.3661x over previous
import functools

import numpy as np
import jax
import jax.numpy as jnp
from jax import lax
from jax.experimental import pallas as pl
from jax.experimental.pallas import tpu as pltpu
from jax.experimental.pallas import tpu_sc as plsc

N = 100000
E = 3200000
SUPPORT = 0.025
DELTA = 0.1
C0 = float(10.0 * np.sqrt(2.0 * 9.81 * 0.3))
EPS = float(SUPPORT ** 2 * 0.1)
REST_DENSITY = 1000.0
WEND_S = float((7.0 / np.pi) / SUPPORT ** 3)
OUT_SCALE = float(SUPPORT * DELTA * C0)

LANES = 16
NC = 2
NS = 16
NW = NC * NS
EW = E // NW
B = 2000
CHUNKS = EW // B
NP_PAD = 100352
PER_TILE = NP_PAD // NS
ROWS = NP_PAD // 128
GRID = ROWS // 8


def _mesh():
    return plsc.VectorSubcoreMesh(
        core_axis_name="c", subcore_axis_name="s", num_cores=NC, num_subcores=NS
    )


def _zero_accs(zbuf, accs, tile_base):
    @pl.loop(0, PER_TILE // LANES)
    def _(k):
        zbuf[pl.ds(k * LANES, LANES)] = jnp.zeros((LANES,), jnp.float32)

    for acc in accs:
        pltpu.sync_copy(zbuf, acc.at[pl.ds(tile_base, PER_TILE)])


def _dump_accs(zbuf, accs, outs, tile_base, out_base):
    for acc, oh in zip(accs, outs):
        pltpu.sync_copy(acc.at[pl.ds(tile_base, PER_TILE)], zbuf)
        pltpu.sync_copy(zbuf, oh.at[pl.ds(out_base, PER_TILE)])


def _wait_all(descs):
    for d in descs:
        d.wait()


def _gradw(q, dxv, dyv):
    t = 1.0 - q
    gwc = WEND_S * ((-20.0 * q) * ((t * t) * t))
    return gwc * dxv, gwc * dyv


def _sweep_one(ii_h, jj_h, dx_h, dy_h, rad_h, px_h, py_h, vol_h, den_h,
               o00_h, o01_h, o10_h, o11_h, osx_h, osy_h, ogx_h, ogy_h,
               ii, jj, dx, dy, rad, xi, yi, xj, yj, di, dj, vj,
               c00, c01, c10, c11, csx, csy, cgx, cgy, zbuf,
               a00, a01, a10, a11, asx, asy, agx, agy, sem):
    c = lax.axis_index("c")
    s = lax.axis_index("s")
    wid = s * NC + c
    tile_base = s * PER_TILE
    accs = (a00, a01, a10, a11, asx, asy, agx, agy)
    _zero_accs(zbuf, accs, tile_base)
    plsc.subcore_barrier()

    base = wid * EW

    @pl.loop(0, CHUNKS)
    def _(ch):
        sl = pl.ds(base + ch * B, B)
        _wait_all([pltpu.async_copy(h.at[sl], v, sem)
                   for h, v in ((ii_h, ii), (jj_h, jj), (dx_h, dx),
                                (dy_h, dy), (rad_h, rad))])
        _wait_all([pltpu.async_copy(px_h.at[ii], xi, sem),
                   pltpu.async_copy(py_h.at[ii], yi, sem),
                   pltpu.async_copy(px_h.at[jj], xj, sem),
                   pltpu.async_copy(py_h.at[jj], yj, sem),
                   pltpu.async_copy(den_h.at[ii], di, sem),
                   pltpu.async_copy(den_h.at[jj], dj, sem),
                   pltpu.async_copy(vol_h.at[jj], vj, sem)])

        @pl.loop(0, B // LANES)
        def _(k):
            v16 = pl.ds(k * LANES, LANES)
            gx, gy = _gradw(rad[v16], dx[v16], dy[v16])
            rx = xj[v16] - xi[v16]
            ry = yj[v16] - yi[v16]
            v = vj[v16]
            drho = dj[v16] * REST_DENSITY - di[v16] * REST_DENSITY
            coef = (drho * v) * 2.0
            c00[v16] = (rx * gx) * v
            c01[v16] = (rx * gy) * v
            c10[v16] = (ry * gx) * v
            c11[v16] = (ry * gy) * v
            csx[v16] = coef * gx
            csy[v16] = coef * gy
            cgx[v16] = gx * v
            cgy[v16] = gy * v

        pltpu.sync_copy(c00, a00.at[ii], add=True)
        pltpu.sync_copy(c01, a01.at[ii], add=True)
        pltpu.sync_copy(c10, a10.at[ii], add=True)
        pltpu.sync_copy(c11, a11.at[ii], add=True)
        pltpu.sync_copy(csx, asx.at[ii], add=True)
        pltpu.sync_copy(csy, asy.at[ii], add=True)
        pltpu.sync_copy(cgx, agx.at[ii], add=True)
        pltpu.sync_copy(cgy, agy.at[ii], add=True)

    plsc.subcore_barrier()
    _dump_accs(zbuf, accs,
               (o00_h, o01_h, o10_h, o11_h, osx_h, osy_h, ogx_h, ogy_h),
               tile_base, c * NP_PAD + tile_base)


def _sweep_two(ii_h, jj_h, dx_h, dy_h, rad_h, px_h, py_h, den_h, vol_h,
               rgx_h, rgy_h, od_h,
               ii, jj, dx, dy, rad, xi, yi, xj, yj, di, dj, vj,
               rgxj, rgyj, cd, zbuf, ad, sem):
    c = lax.axis_index("c")
    s = lax.axis_index("s")
    wid = s * NC + c
    tile_base = s * PER_TILE
    _zero_accs(zbuf, (ad,), tile_base)
    plsc.subcore_barrier()

    base = wid * EW

    @pl.loop(0, CHUNKS)
    def _(ch):
        sl = pl.ds(base + ch * B, B)
        _wait_all([pltpu.async_copy(h.at[sl], v, sem)
                   for h, v in ((ii_h, ii), (jj_h, jj), (dx_h, dx),
                                (dy_h, dy), (rad_h, rad))])
        _wait_all([pltpu.async_copy(px_h.at[ii], xi, sem),
                   pltpu.async_copy(py_h.at[ii], yi, sem),
                   pltpu.async_copy(px_h.at[jj], xj, sem),
                   pltpu.async_copy(py_h.at[jj], yj, sem),
                   pltpu.async_copy(den_h.at[ii], di, sem),
                   pltpu.async_copy(den_h.at[jj], dj, sem),
                   pltpu.async_copy(vol_h.at[jj], vj, sem),
                   pltpu.async_copy(rgx_h.at[jj], rgxj, sem),
                   pltpu.async_copy(rgy_h.at[jj], rgyj, sem)])

        @pl.loop(0, B // LANES)
        def _(k):
            v16 = pl.ds(k * LANES, LANES)
            gx, gy = _gradw(rad[v16], dx[v16], dy[v16])
            rx = xj[v16] - xi[v16]
            ry = yj[v16] - yi[v16]
            r2 = (rx * rx + ry * ry) + EPS
            drho = dj[v16] * REST_DENSITY - di[v16] * REST_DENSITY
            sc = (2.0 * drho) / r2
            cd[v16] = ((sc * rx - rgxj[v16]) * gx
                       + (sc * ry - rgyj[v16]) * gy) * vj[v16]

        pltpu.sync_copy(cd, ad.at[ii], add=True)

    plsc.subcore_barrier()
    _dump_accs(zbuf, (ad,), (od_h,), tile_base, c * NP_PAD + tile_base)


def _pinv_rg_body(n00a, n00b, n01a, n01b, n10a, n10b, n11a, n11b,
                  sxa, sxb, sya, syb, gxa, gxb, gya, gyb,
                  rgx, rgy, gvx, gvy):
    m00 = n00a[...] + n00b[...]
    m01 = n01a[...] + n01b[...]
    m10 = n10a[...] + n10b[...]
    m11 = n11a[...] + n11b[...]
    sx = sxa[...] + sxb[...]
    sy = sya[...] + syb[...]
    det = m00 * m11 - m01 * m10
    dets = jnp.where(jnp.abs(det) > 1e-7, det,
                     jnp.where(det >= 0, 1e-7, -1e-7))
    l00 = m11 / dets
    l01 = -m01 / dets
    l10 = -m10 / dets
    l11 = m00 / dets
    rgx[...] = -(l00 * sx + l01 * sy)
    rgy[...] = -(l10 * sx + l11 * sy)
    gvx[...] = gxa[...] + gxb[...]
    gvy[...] = gya[...] + gyb[...]


def _final_body(da, db, rgx, rgy, gvx, gvy, o):
    o[...] = ((da[...] + db[...])
              - (rgx[...] * gvx[...] + rgy[...] * gvy[...])) * OUT_SCALE


def _tc_call(body, n_in, n_out, args):
    bs = pl.BlockSpec((8, 128), lambda g: (g, 0))
    shape = jax.ShapeDtypeStruct((ROWS, 128), jnp.float32)
    f = pl.pallas_call(
        body,
        grid=(GRID,),
        in_specs=[bs] * n_in,
        out_specs=[bs] * n_out if n_out > 1 else bs,
        out_shape=[shape] * n_out if n_out > 1 else shape,
    )
    return f(*args)


def _split2(flat):
    r = flat.reshape(2, ROWS, 128)
    return r[0], r[1]


def kernel(positions, volumes, distances, radialDistances, density, i, j):
    ii = i.astype(jnp.int32)
    jj = j.astype(jnp.int32)
    pad = NP_PAD - N
    px = jnp.pad(positions[:, 0], (0, pad))
    py = jnp.pad(positions[:, 1], (0, pad))
    vol = jnp.pad(volumes, (0, pad))
    den = jnp.pad(density, (0, pad))
    dx = distances[:, 0]
    dy = distances[:, 1]
    rad = radialDistances

    mesh = _mesh()
    ebuf = lambda: pltpu.VMEM((B,), jnp.float32)
    eibuf = lambda: pltpu.VMEM((B,), jnp.int32)
    zbuf = pltpu.VMEM((PER_TILE,), jnp.float32)
    acc = lambda: pltpu.VMEM_SHARED((NP_PAD,), jnp.float32)
    oshape = jax.ShapeDtypeStruct((2 * NP_PAD,), jnp.float32)

    sweep1 = pl.kernel(
        _sweep_one,
        out_type=(oshape,) * 8,
        mesh=mesh,
        scratch_types=(
            [eibuf(), eibuf()] + [ebuf()] * 18
            + [zbuf] + [acc() for _ in range(8)] + [pltpu.SemaphoreType.DMA]
        ),
    )
    nm00, nm01, nm10, nm11, sxp, syp, gxp, gyp = sweep1(
        ii, jj, dx, dy, rad, px, py, vol, den)

    parts = (_split2(nm00) + _split2(nm01) + _split2(nm10) + _split2(nm11)
             + _split2(sxp) + _split2(syp) + _split2(gxp) + _split2(gyp))
    rgx, rgy, gvx, gvy = _tc_call(_pinv_rg_body, 16, 4, parts)
    rgx_f = rgx.reshape(NP_PAD)
    rgy_f = rgy.reshape(NP_PAD)

    sweep2 = pl.kernel(
        _sweep_two,
        out_type=oshape,
        mesh=mesh,
        scratch_types=(
            [eibuf(), eibuf()] + [ebuf()] * 13
            + [zbuf, acc(), pltpu.SemaphoreType.DMA]
        ),
    )
    dd2 = sweep2(ii, jj, dx, dy, rad, px, py, den, vol, rgx_f, rgy_f)

    out = _tc_call(_final_body, 6, 1,
                   _split2(dd2) + (rgx, rgy, gvx, gvy)).reshape(NP_PAD)
    return out[:N]


# speedup vs baseline: 467.8423x; 2.1814x over previous
import functools

import numpy as np
import jax
import jax.numpy as jnp
from jax import lax
from jax.experimental import pallas as pl
from jax.experimental.pallas import tpu as pltpu
from jax.experimental.pallas import tpu_sc as plsc

N = 100000
E = 3200000
SUPPORT = 0.025
DELTA = 0.1
C0 = float(10.0 * np.sqrt(2.0 * 9.81 * 0.3))
EPS = float(SUPPORT ** 2 * 0.1)
REST_DENSITY = 1000.0
WEND_S = float((7.0 / np.pi) / SUPPORT ** 3)
OUT_SCALE = float(SUPPORT * DELTA * C0)

LANES = 16
NC = 2
NS = 16
NW = NC * NS
EW = E // NW
B = 2000
CHUNKS = EW // B
NP_PAD = 100352
PER_TILE = NP_PAD // NS
ROWS = NP_PAD // 128
GRID = ROWS // 8


def _mesh():
    return plsc.VectorSubcoreMesh(
        core_axis_name="c", subcore_axis_name="s", num_cores=NC, num_subcores=NS
    )


def _stage_tables(zbuf, tables, tile_base):
    sl = pl.ds(tile_base, PER_TILE)
    for hbm, sh in tables:
        pltpu.sync_copy(hbm.at[sl], zbuf)
        pltpu.sync_copy(zbuf, sh.at[sl])


def _zero_accs(zbuf, accs, tile_base):
    @pl.loop(0, PER_TILE // LANES)
    def _(k):
        zbuf[pl.ds(k * LANES, LANES)] = jnp.zeros((LANES,), jnp.float32)

    for acc in accs:
        pltpu.sync_copy(zbuf, acc.at[pl.ds(tile_base, PER_TILE)])


def _dump_accs(zbuf, accs, outs, tile_base, out_base):
    for acc, oh in zip(accs, outs):
        pltpu.sync_copy(acc.at[pl.ds(tile_base, PER_TILE)], zbuf)
        pltpu.sync_copy(zbuf, oh.at[pl.ds(out_base, PER_TILE)])


def _wait_all(descs):
    for d in descs:
        d.wait()


def _gradw(q, dxv, dyv):
    t = 1.0 - q
    gwc = WEND_S * ((-20.0 * q) * ((t * t) * t))
    return gwc * dxv, gwc * dyv


def _sweep_one(ii_h, jj_h, dx_h, dy_h, rad_h, px_h, py_h, vol_h, den_h,
               o00_h, o01_h, o10_h, o11_h, osx_h, osy_h, ogx_h, ogy_h,
               ii, jj, dx, dy, rad, xi, yi, xj, yj, di, dj, vj,
               c00, c01, c10, c11, csx, csy, cgx, cgy, zbuf,
               tpx, tpy, tvol, tden,
               a00, a01, a10, a11, asx, asy, agx, agy, sem):
    c = lax.axis_index("c")
    s = lax.axis_index("s")
    wid = s * NC + c
    tile_base = s * PER_TILE
    accs = (a00, a01, a10, a11, asx, asy, agx, agy)
    _stage_tables(zbuf, ((px_h, tpx), (py_h, tpy), (vol_h, tvol),
                         (den_h, tden)), tile_base)
    _zero_accs(zbuf, accs, tile_base)
    plsc.subcore_barrier()

    base = wid * EW

    @pl.loop(0, CHUNKS)
    def _(ch):
        sl = pl.ds(base + ch * B, B)
        _wait_all([pltpu.async_copy(h.at[sl], v, sem)
                   for h, v in ((ii_h, ii), (jj_h, jj), (dx_h, dx),
                                (dy_h, dy), (rad_h, rad))])
        _wait_all([pltpu.async_copy(tpx.at[ii], xi, sem),
                   pltpu.async_copy(tpy.at[ii], yi, sem),
                   pltpu.async_copy(tpx.at[jj], xj, sem),
                   pltpu.async_copy(tpy.at[jj], yj, sem),
                   pltpu.async_copy(tden.at[ii], di, sem),
                   pltpu.async_copy(tden.at[jj], dj, sem),
                   pltpu.async_copy(tvol.at[jj], vj, sem)])

        @pl.loop(0, B // LANES)
        def _(k):
            v16 = pl.ds(k * LANES, LANES)
            gx, gy = _gradw(rad[v16], dx[v16], dy[v16])
            rx = xj[v16] - xi[v16]
            ry = yj[v16] - yi[v16]
            v = vj[v16]
            drho = dj[v16] * REST_DENSITY - di[v16] * REST_DENSITY
            coef = (drho * v) * 2.0
            c00[v16] = (rx * gx) * v
            c01[v16] = (rx * gy) * v
            c10[v16] = (ry * gx) * v
            c11[v16] = (ry * gy) * v
            csx[v16] = coef * gx
            csy[v16] = coef * gy
            cgx[v16] = gx * v
            cgy[v16] = gy * v

        pltpu.sync_copy(c00, a00.at[ii], add=True)
        pltpu.sync_copy(c01, a01.at[ii], add=True)
        pltpu.sync_copy(c10, a10.at[ii], add=True)
        pltpu.sync_copy(c11, a11.at[ii], add=True)
        pltpu.sync_copy(csx, asx.at[ii], add=True)
        pltpu.sync_copy(csy, asy.at[ii], add=True)
        pltpu.sync_copy(cgx, agx.at[ii], add=True)
        pltpu.sync_copy(cgy, agy.at[ii], add=True)

    plsc.subcore_barrier()
    _dump_accs(zbuf, accs,
               (o00_h, o01_h, o10_h, o11_h, osx_h, osy_h, ogx_h, ogy_h),
               tile_base, c * NP_PAD + tile_base)


def _sweep_two(ii_h, jj_h, dx_h, dy_h, rad_h, px_h, py_h, den_h, vol_h,
               rgx_h, rgy_h, od_h,
               ii, jj, dx, dy, rad, xi, yi, xj, yj, di, dj, vj,
               rgxj, rgyj, cd, zbuf,
               tpx, tpy, tden, tvol, trgx, trgy, ad, sem):
    c = lax.axis_index("c")
    s = lax.axis_index("s")
    wid = s * NC + c
    tile_base = s * PER_TILE
    _stage_tables(zbuf, ((px_h, tpx), (py_h, tpy), (den_h, tden),
                         (vol_h, tvol), (rgx_h, trgx), (rgy_h, trgy)),
                  tile_base)
    _zero_accs(zbuf, (ad,), tile_base)
    plsc.subcore_barrier()

    base = wid * EW

    @pl.loop(0, CHUNKS)
    def _(ch):
        sl = pl.ds(base + ch * B, B)
        _wait_all([pltpu.async_copy(h.at[sl], v, sem)
                   for h, v in ((ii_h, ii), (jj_h, jj), (dx_h, dx),
                                (dy_h, dy), (rad_h, rad))])
        _wait_all([pltpu.async_copy(tpx.at[ii], xi, sem),
                   pltpu.async_copy(tpy.at[ii], yi, sem),
                   pltpu.async_copy(tpx.at[jj], xj, sem),
                   pltpu.async_copy(tpy.at[jj], yj, sem),
                   pltpu.async_copy(tden.at[ii], di, sem),
                   pltpu.async_copy(tden.at[jj], dj, sem),
                   pltpu.async_copy(tvol.at[jj], vj, sem),
                   pltpu.async_copy(trgx.at[jj], rgxj, sem),
                   pltpu.async_copy(trgy.at[jj], rgyj, sem)])

        @pl.loop(0, B // LANES)
        def _(k):
            v16 = pl.ds(k * LANES, LANES)
            gx, gy = _gradw(rad[v16], dx[v16], dy[v16])
            rx = xj[v16] - xi[v16]
            ry = yj[v16] - yi[v16]
            r2 = (rx * rx + ry * ry) + EPS
            drho = dj[v16] * REST_DENSITY - di[v16] * REST_DENSITY
            sc = (2.0 * drho) / r2
            cd[v16] = ((sc * rx - rgxj[v16]) * gx
                       + (sc * ry - rgyj[v16]) * gy) * vj[v16]

        pltpu.sync_copy(cd, ad.at[ii], add=True)

    plsc.subcore_barrier()
    _dump_accs(zbuf, (ad,), (od_h,), tile_base, c * NP_PAD + tile_base)


def _pinv_rg_body(n00a, n00b, n01a, n01b, n10a, n10b, n11a, n11b,
                  sxa, sxb, sya, syb, gxa, gxb, gya, gyb,
                  rgx, rgy, gvx, gvy):
    m00 = n00a[...] + n00b[...]
    m01 = n01a[...] + n01b[...]
    m10 = n10a[...] + n10b[...]
    m11 = n11a[...] + n11b[...]
    sx = sxa[...] + sxb[...]
    sy = sya[...] + syb[...]
    det = m00 * m11 - m01 * m10
    dets = jnp.where(jnp.abs(det) > 1e-7, det,
                     jnp.where(det >= 0, 1e-7, -1e-7))
    l00 = m11 / dets
    l01 = -m01 / dets
    l10 = -m10 / dets
    l11 = m00 / dets
    rgx[...] = -(l00 * sx + l01 * sy)
    rgy[...] = -(l10 * sx + l11 * sy)
    gvx[...] = gxa[...] + gxb[...]
    gvy[...] = gya[...] + gyb[...]


def _final_body(da, db, rgx, rgy, gvx, gvy, o):
    o[...] = ((da[...] + db[...])
              - (rgx[...] * gvx[...] + rgy[...] * gvy[...])) * OUT_SCALE


def _tc_call(body, n_in, n_out, args):
    bs = pl.BlockSpec((8, 128), lambda g: (g, 0))
    shape = jax.ShapeDtypeStruct((ROWS, 128), jnp.float32)
    f = pl.pallas_call(
        body,
        grid=(GRID,),
        in_specs=[bs] * n_in,
        out_specs=[bs] * n_out if n_out > 1 else bs,
        out_shape=[shape] * n_out if n_out > 1 else shape,
    )
    return f(*args)


def _split2(flat):
    r = flat.reshape(2, ROWS, 128)
    return r[0], r[1]


def kernel(positions, volumes, distances, radialDistances, density, i, j):
    ii = i.astype(jnp.int32)
    jj = j.astype(jnp.int32)
    pad = NP_PAD - N
    px = jnp.pad(positions[:, 0], (0, pad))
    py = jnp.pad(positions[:, 1], (0, pad))
    vol = jnp.pad(volumes, (0, pad))
    den = jnp.pad(density, (0, pad))
    dx = distances[:, 0]
    dy = distances[:, 1]
    rad = radialDistances

    mesh = _mesh()
    ebuf = lambda: pltpu.VMEM((B,), jnp.float32)
    eibuf = lambda: pltpu.VMEM((B,), jnp.int32)
    zbuf = pltpu.VMEM((PER_TILE,), jnp.float32)
    shr = lambda: pltpu.VMEM_SHARED((NP_PAD,), jnp.float32)
    oshape = jax.ShapeDtypeStruct((2 * NP_PAD,), jnp.float32)
    cparams = pltpu.CompilerParams(use_tc_tiling_on_sc=False)

    sweep1 = pl.kernel(
        _sweep_one,
        out_type=(oshape,) * 8,
        mesh=mesh,
        compiler_params=cparams,
        scratch_types=(
            [eibuf(), eibuf()] + [ebuf()] * 18
            + [zbuf] + [shr() for _ in range(12)] + [pltpu.SemaphoreType.DMA]
        ),
    )
    nm00, nm01, nm10, nm11, sxp, syp, gxp, gyp = sweep1(
        ii, jj, dx, dy, rad, px, py, vol, den)

    parts = (_split2(nm00) + _split2(nm01) + _split2(nm10) + _split2(nm11)
             + _split2(sxp) + _split2(syp) + _split2(gxp) + _split2(gyp))
    rgx, rgy, gvx, gvy = _tc_call(_pinv_rg_body, 16, 4, parts)
    rgx_f = rgx.reshape(NP_PAD)
    rgy_f = rgy.reshape(NP_PAD)

    sweep2 = pl.kernel(
        _sweep_two,
        out_type=oshape,
        mesh=mesh,
        compiler_params=cparams,
        scratch_types=(
            [eibuf(), eibuf()] + [ebuf()] * 13
            + [zbuf] + [shr() for _ in range(7)] + [pltpu.SemaphoreType.DMA]
        ),
    )
    dd2 = sweep2(ii, jj, dx, dy, rad, px, py, den, vol, rgx_f, rgy_f)

    out = _tc_call(_final_body, 6, 1,
                   _split2(dd2) + (rgx, rgy, gvx, gvy)).reshape(NP_PAD)
    return out[:N]


# speedup vs baseline: 472.7271x; 1.0104x over previous
import functools

import numpy as np
import jax
import jax.numpy as jnp
from jax import lax
from jax.experimental import pallas as pl
from jax.experimental.pallas import tpu as pltpu
from jax.experimental.pallas import tpu_sc as plsc

N = 100000
E = 3200000
SUPPORT = 0.025
DELTA = 0.1
C0 = float(10.0 * np.sqrt(2.0 * 9.81 * 0.3))
EPS = float(SUPPORT ** 2 * 0.1)
REST_DENSITY = 1000.0
WEND_S = float((7.0 / np.pi) / SUPPORT ** 3)
OUT_SCALE = float(SUPPORT * DELTA * C0)

LANES = 16
NC = 2
NS = 16
NW = NC * NS
EW = E // NW
B = 2000
CHUNKS = EW // B
NP_PAD = 100352
PER_TILE = NP_PAD // NS
ROWS = NP_PAD // 128
GRID = ROWS // 8


def _mesh():
    return plsc.VectorSubcoreMesh(
        core_axis_name="c", subcore_axis_name="s", num_cores=NC, num_subcores=NS
    )


def _stage_tables(zbuf, tables, tile_base):
    sl = pl.ds(tile_base, PER_TILE)
    for hbm, sh in tables:
        pltpu.sync_copy(hbm.at[sl], zbuf)
        pltpu.sync_copy(zbuf, sh.at[sl])


def _zero_accs(zbuf, accs, tile_base):
    @pl.loop(0, PER_TILE // LANES)
    def _(k):
        zbuf[pl.ds(k * LANES, LANES)] = jnp.zeros((LANES,), jnp.float32)

    for acc in accs:
        pltpu.sync_copy(zbuf, acc.at[pl.ds(tile_base, PER_TILE)])


def _dump_accs(zbuf, accs, outs, tile_base, out_base):
    for acc, oh in zip(accs, outs):
        pltpu.sync_copy(acc.at[pl.ds(tile_base, PER_TILE)], zbuf)
        pltpu.sync_copy(zbuf, oh.at[pl.ds(out_base, PER_TILE)])


def _wait_all(descs):
    for d in descs:
        d.wait()


def _gradw(q, dxv, dyv):
    t = 1.0 - q
    gwc = WEND_S * ((-20.0 * q) * ((t * t) * t))
    return gwc * dxv, gwc * dyv


def _sweep_one(ii_h, jj_h, dx_h, dy_h, rad_h, px_h, py_h, vol_h, den_h,
               o00_h, o01_h, o10_h, o11_h, osx_h, osy_h, ogx_h, ogy_h,
               ii, jj, dx, dy, rad, xi, yi, xj, yj, di, dj, vj,
               c00, c01, c10, c11, csx, csy, cgx, cgy, zbuf,
               tpx, tpy, tvol, tden,
               a00, a01, a10, a11, asx, asy, agx, agy, sem):
    c = lax.axis_index("c")
    s = lax.axis_index("s")
    wid = s * NC + c
    tile_base = s * PER_TILE
    accs = (a00, a01, a10, a11, asx, asy, agx, agy)
    _stage_tables(zbuf, ((px_h, tpx), (py_h, tpy), (vol_h, tvol),
                         (den_h, tden)), tile_base)
    _zero_accs(zbuf, accs, tile_base)
    plsc.subcore_barrier()

    base = wid * EW

    @pl.loop(0, CHUNKS)
    def _(ch):
        sl = pl.ds(base + ch * B, B)
        _wait_all([pltpu.async_copy(h.at[sl], v, sem)
                   for h, v in ((ii_h, ii), (jj_h, jj), (dx_h, dx),
                                (dy_h, dy), (rad_h, rad))])
        _wait_all([pltpu.async_copy(tpx.at[ii], xi, sem),
                   pltpu.async_copy(tpy.at[ii], yi, sem),
                   pltpu.async_copy(tpx.at[jj], xj, sem),
                   pltpu.async_copy(tpy.at[jj], yj, sem),
                   pltpu.async_copy(tden.at[ii], di, sem),
                   pltpu.async_copy(tden.at[jj], dj, sem),
                   pltpu.async_copy(tvol.at[jj], vj, sem)])

        @pl.loop(0, B // LANES)
        def _(k):
            v16 = pl.ds(k * LANES, LANES)
            gx, gy = _gradw(rad[v16], dx[v16], dy[v16])
            rx = xj[v16] - xi[v16]
            ry = yj[v16] - yi[v16]
            v = vj[v16]
            drho = dj[v16] * REST_DENSITY - di[v16] * REST_DENSITY
            coef = (drho * v) * 2.0
            c00[v16] = (rx * gx) * v
            c01[v16] = (rx * gy) * v
            c10[v16] = (ry * gx) * v
            c11[v16] = (ry * gy) * v
            csx[v16] = coef * gx
            csy[v16] = coef * gy
            cgx[v16] = gx * v
            cgy[v16] = gy * v

        _wait_all([pltpu.async_copy(c00, a00.at[ii], sem, add=True),
                   pltpu.async_copy(c01, a01.at[ii], sem, add=True),
                   pltpu.async_copy(c10, a10.at[ii], sem, add=True),
                   pltpu.async_copy(c11, a11.at[ii], sem, add=True),
                   pltpu.async_copy(csx, asx.at[ii], sem, add=True),
                   pltpu.async_copy(csy, asy.at[ii], sem, add=True),
                   pltpu.async_copy(cgx, agx.at[ii], sem, add=True),
                   pltpu.async_copy(cgy, agy.at[ii], sem, add=True)])

    plsc.subcore_barrier()
    _dump_accs(zbuf, accs,
               (o00_h, o01_h, o10_h, o11_h, osx_h, osy_h, ogx_h, ogy_h),
               tile_base, c * NP_PAD + tile_base)


def _sweep_two(ii_h, jj_h, dx_h, dy_h, rad_h, px_h, py_h, den_h, vol_h,
               rgx_h, rgy_h, od_h,
               ii, jj, dx, dy, rad, xi, yi, xj, yj, di, dj, vj,
               rgxj, rgyj, cd, zbuf,
               tpx, tpy, tden, tvol, trgx, trgy, ad, sem):
    c = lax.axis_index("c")
    s = lax.axis_index("s")
    wid = s * NC + c
    tile_base = s * PER_TILE
    _stage_tables(zbuf, ((px_h, tpx), (py_h, tpy), (den_h, tden),
                         (vol_h, tvol), (rgx_h, trgx), (rgy_h, trgy)),
                  tile_base)
    _zero_accs(zbuf, (ad,), tile_base)
    plsc.subcore_barrier()

    base = wid * EW

    @pl.loop(0, CHUNKS)
    def _(ch):
        sl = pl.ds(base + ch * B, B)
        _wait_all([pltpu.async_copy(h.at[sl], v, sem)
                   for h, v in ((ii_h, ii), (jj_h, jj), (dx_h, dx),
                                (dy_h, dy), (rad_h, rad))])
        _wait_all([pltpu.async_copy(tpx.at[ii], xi, sem),
                   pltpu.async_copy(tpy.at[ii], yi, sem),
                   pltpu.async_copy(tpx.at[jj], xj, sem),
                   pltpu.async_copy(tpy.at[jj], yj, sem),
                   pltpu.async_copy(tden.at[ii], di, sem),
                   pltpu.async_copy(tden.at[jj], dj, sem),
                   pltpu.async_copy(tvol.at[jj], vj, sem),
                   pltpu.async_copy(trgx.at[jj], rgxj, sem),
                   pltpu.async_copy(trgy.at[jj], rgyj, sem)])

        @pl.loop(0, B // LANES)
        def _(k):
            v16 = pl.ds(k * LANES, LANES)
            gx, gy = _gradw(rad[v16], dx[v16], dy[v16])
            rx = xj[v16] - xi[v16]
            ry = yj[v16] - yi[v16]
            r2 = (rx * rx + ry * ry) + EPS
            drho = dj[v16] * REST_DENSITY - di[v16] * REST_DENSITY
            sc = (2.0 * drho) / r2
            cd[v16] = ((sc * rx - rgxj[v16]) * gx
                       + (sc * ry - rgyj[v16]) * gy) * vj[v16]

        pltpu.sync_copy(cd, ad.at[ii], add=True)

    plsc.subcore_barrier()
    _dump_accs(zbuf, (ad,), (od_h,), tile_base, c * NP_PAD + tile_base)


def _pinv_rg_body(n00a, n00b, n01a, n01b, n10a, n10b, n11a, n11b,
                  sxa, sxb, sya, syb, gxa, gxb, gya, gyb,
                  rgx, rgy, gvx, gvy):
    m00 = n00a[...] + n00b[...]
    m01 = n01a[...] + n01b[...]
    m10 = n10a[...] + n10b[...]
    m11 = n11a[...] + n11b[...]
    sx = sxa[...] + sxb[...]
    sy = sya[...] + syb[...]
    det = m00 * m11 - m01 * m10
    dets = jnp.where(jnp.abs(det) > 1e-7, det,
                     jnp.where(det >= 0, 1e-7, -1e-7))
    l00 = m11 / dets
    l01 = -m01 / dets
    l10 = -m10 / dets
    l11 = m00 / dets
    rgx[...] = -(l00 * sx + l01 * sy)
    rgy[...] = -(l10 * sx + l11 * sy)
    gvx[...] = gxa[...] + gxb[...]
    gvy[...] = gya[...] + gyb[...]


def _final_body(da, db, rgx, rgy, gvx, gvy, o):
    o[...] = ((da[...] + db[...])
              - (rgx[...] * gvx[...] + rgy[...] * gvy[...])) * OUT_SCALE


def _tc_call(body, n_in, n_out, args):
    bs = pl.BlockSpec((8, 128), lambda g: (g, 0))
    shape = jax.ShapeDtypeStruct((ROWS, 128), jnp.float32)
    f = pl.pallas_call(
        body,
        grid=(GRID,),
        in_specs=[bs] * n_in,
        out_specs=[bs] * n_out if n_out > 1 else bs,
        out_shape=[shape] * n_out if n_out > 1 else shape,
    )
    return f(*args)


def _split2(flat):
    r = flat.reshape(2, ROWS, 128)
    return r[0], r[1]


def kernel(positions, volumes, distances, radialDistances, density, i, j):
    ii = i.astype(jnp.int32)
    jj = j.astype(jnp.int32)
    pad = NP_PAD - N
    px = jnp.pad(positions[:, 0], (0, pad))
    py = jnp.pad(positions[:, 1], (0, pad))
    vol = jnp.pad(volumes, (0, pad))
    den = jnp.pad(density, (0, pad))
    dx = distances[:, 0]
    dy = distances[:, 1]
    rad = radialDistances

    mesh = _mesh()
    ebuf = lambda: pltpu.VMEM((B,), jnp.float32)
    eibuf = lambda: pltpu.VMEM((B,), jnp.int32)
    zbuf = pltpu.VMEM((PER_TILE,), jnp.float32)
    shr = lambda: pltpu.VMEM_SHARED((NP_PAD,), jnp.float32)
    oshape = jax.ShapeDtypeStruct((2 * NP_PAD,), jnp.float32)
    cparams = pltpu.CompilerParams(use_tc_tiling_on_sc=False)

    sweep1 = pl.kernel(
        _sweep_one,
        out_type=(oshape,) * 8,
        mesh=mesh,
        compiler_params=cparams,
        scratch_types=(
            [eibuf(), eibuf()] + [ebuf()] * 18
            + [zbuf] + [shr() for _ in range(12)] + [pltpu.SemaphoreType.DMA]
        ),
    )
    nm00, nm01, nm10, nm11, sxp, syp, gxp, gyp = sweep1(
        ii, jj, dx, dy, rad, px, py, vol, den)

    parts = (_split2(nm00) + _split2(nm01) + _split2(nm10) + _split2(nm11)
             + _split2(sxp) + _split2(syp) + _split2(gxp) + _split2(gyp))
    rgx, rgy, gvx, gvy = _tc_call(_pinv_rg_body, 16, 4, parts)
    rgx_f = rgx.reshape(NP_PAD)
    rgy_f = rgy.reshape(NP_PAD)

    sweep2 = pl.kernel(
        _sweep_two,
        out_type=oshape,
        mesh=mesh,
        compiler_params=cparams,
        scratch_types=(
            [eibuf(), eibuf()] + [ebuf()] * 13
            + [zbuf] + [shr() for _ in range(7)] + [pltpu.SemaphoreType.DMA]
        ),
    )
    dd2 = sweep2(ii, jj, dx, dy, rad, px, py, den, vol, rgx_f, rgy_f)

    out = _tc_call(_final_body, 6, 1,
                   _split2(dd2) + (rgx, rgy, gvx, gvy)).reshape(NP_PAD)
    return out[:N]


# speedup vs baseline: 565.9124x; 1.1971x over previous
import functools

import numpy as np
import jax
import jax.numpy as jnp
from jax import lax
from jax.experimental import pallas as pl
from jax.experimental.pallas import tpu as pltpu
from jax.experimental.pallas import tpu_sc as plsc

N = 100000
E = 3200000
SUPPORT = 0.025
DELTA = 0.1
C0 = float(10.0 * np.sqrt(2.0 * 9.81 * 0.3))
EPS = float(SUPPORT ** 2 * 0.1)
REST_DENSITY = 1000.0
WEND_S = float((7.0 / np.pi) / SUPPORT ** 3)
OUT_SCALE = float(SUPPORT * DELTA * C0)

LANES = 16
NC = 2
NS = 16
NW = NC * NS
EW = E // NW
B = 2000
CHUNKS = EW // B
NP_PAD = 100352
PER_TILE = NP_PAD // NS
ROWS = NP_PAD // 128
GRID = ROWS // 8


def _mesh():
    return plsc.VectorSubcoreMesh(
        core_axis_name="c", subcore_axis_name="s", num_cores=NC, num_subcores=NS
    )


def _stage_tables(zbuf, tables, tile_base):
    sl = pl.ds(tile_base, PER_TILE)
    for hbm, sh in tables:
        pltpu.sync_copy(hbm.at[sl], zbuf)
        pltpu.sync_copy(zbuf, sh.at[sl])


def _zero_accs(zbuf, accs, tile_base):
    @pl.loop(0, PER_TILE // LANES)
    def _(k):
        zbuf[pl.ds(k * LANES, LANES)] = jnp.zeros((LANES,), jnp.float32)

    for acc in accs:
        pltpu.sync_copy(zbuf, acc.at[pl.ds(tile_base, PER_TILE)])


def _dump_accs(zbuf, accs, outs, tile_base, out_base):
    for acc, oh in zip(accs, outs):
        pltpu.sync_copy(acc.at[pl.ds(tile_base, PER_TILE)], zbuf)
        pltpu.sync_copy(zbuf, oh.at[pl.ds(out_base, PER_TILE)])


def _wait_all(descs):
    for d in descs:
        d.wait()


def _sweep_one(ii_h, jj_h, dx_h, dy_h, rad_h, px_h, py_h, vol_h, den_h,
               o00_h, o01_h, o10_h, o11_h, osx_h, osy_h,
               egx_h, egy_h, erx_h, ery_h, ecf_h, evj_h,
               ii, jj, dx, dy, rad, xi, yi, xj, yj, di, dj, vj,
               c00, c01, c10, c11, csx, csy, sgx, sgy, srx, sry, scf, zbuf,
               tpx, tpy, tvol, tden,
               a00, a01, a10, a11, asx, asy, sem, sem2):
    c = lax.axis_index("c")
    s = lax.axis_index("s")
    wid = s * NC + c
    tile_base = s * PER_TILE
    accs = (a00, a01, a10, a11, asx, asy)
    _stage_tables(zbuf, ((px_h, tpx), (py_h, tpy), (vol_h, tvol),
                         (den_h, tden)), tile_base)
    _zero_accs(zbuf, accs, tile_base)
    plsc.subcore_barrier()

    base = wid * EW

    @pl.loop(0, CHUNKS)
    def _(ch):
        sl = pl.ds(base + ch * B, B)
        _wait_all([pltpu.async_copy(h.at[sl], v, sem)
                   for h, v in ((ii_h, ii), (jj_h, jj), (dx_h, dx),
                                (dy_h, dy), (rad_h, rad))])
        _wait_all([pltpu.async_copy(tpx.at[ii], xi, sem),
                   pltpu.async_copy(tpy.at[ii], yi, sem),
                   pltpu.async_copy(tpx.at[jj], xj, sem),
                   pltpu.async_copy(tpy.at[jj], yj, sem),
                   pltpu.async_copy(tden.at[ii], di, sem),
                   pltpu.async_copy(tden.at[jj], dj, sem),
                   pltpu.async_copy(tvol.at[jj], vj, sem)])

        @pl.loop(0, B // LANES)
        def _(k):
            v16 = pl.ds(k * LANES, LANES)
            q = rad[v16]
            t = 1.0 - q
            gwc = WEND_S * ((-20.0 * q) * ((t * t) * t))
            gx = gwc * dx[v16]
            gy = gwc * dy[v16]
            rx = xj[v16] - xi[v16]
            ry = yj[v16] - yi[v16]
            v = vj[v16]
            drho = dj[v16] * REST_DENSITY - di[v16] * REST_DENSITY
            coef = (drho * v) * 2.0
            c00[v16] = (rx * gx) * v
            c01[v16] = (rx * gy) * v
            c10[v16] = (ry * gx) * v
            c11[v16] = (ry * gy) * v
            csx[v16] = coef * gx
            csy[v16] = coef * gy
            sgx[v16] = gx
            sgy[v16] = gy
            srx[v16] = rx
            sry[v16] = ry
            scf[v16] = coef

        _wait_all([pltpu.async_copy(sgx, egx_h.at[sl], sem2),
                   pltpu.async_copy(sgy, egy_h.at[sl], sem2),
                   pltpu.async_copy(srx, erx_h.at[sl], sem2),
                   pltpu.async_copy(sry, ery_h.at[sl], sem2),
                   pltpu.async_copy(scf, ecf_h.at[sl], sem2),
                   pltpu.async_copy(vj, evj_h.at[sl], sem2)])
        _wait_all([pltpu.async_copy(c00, a00.at[ii], sem, add=True),
                   pltpu.async_copy(c01, a01.at[ii], sem, add=True),
                   pltpu.async_copy(c10, a10.at[ii], sem, add=True),
                   pltpu.async_copy(c11, a11.at[ii], sem, add=True),
                   pltpu.async_copy(csx, asx.at[ii], sem, add=True),
                   pltpu.async_copy(csy, asy.at[ii], sem, add=True)])

    plsc.subcore_barrier()
    _dump_accs(zbuf, accs,
               (o00_h, o01_h, o10_h, o11_h, osx_h, osy_h),
               tile_base, c * NP_PAD + tile_base)


def _sweep_two(ii_h, jj_h, egx_h, egy_h, erx_h, ery_h, ecf_h, evj_h,
               rgx_h, rgy_h, od_h, ogx_h, ogy_h,
               ii, jj, gx, gy, rx, ry, cf, vj, rgxj, rgyj, cd, cgx, cgy,
               zbuf, trgx, trgy, ad, agx, agy, sem):
    c = lax.axis_index("c")
    s = lax.axis_index("s")
    wid = s * NC + c
    tile_base = s * PER_TILE
    accs = (ad, agx, agy)
    _stage_tables(zbuf, ((rgx_h, trgx), (rgy_h, trgy)), tile_base)
    _zero_accs(zbuf, accs, tile_base)
    plsc.subcore_barrier()

    base = wid * EW

    @pl.loop(0, CHUNKS)
    def _(ch):
        sl = pl.ds(base + ch * B, B)
        _wait_all([pltpu.async_copy(h.at[sl], v, sem)
                   for h, v in ((ii_h, ii), (jj_h, jj), (egx_h, gx),
                                (egy_h, gy), (erx_h, rx), (ery_h, ry),
                                (ecf_h, cf), (evj_h, vj))])
        _wait_all([pltpu.async_copy(trgx.at[jj], rgxj, sem),
                   pltpu.async_copy(trgy.at[jj], rgyj, sem)])

        @pl.loop(0, B // LANES)
        def _(k):
            v16 = pl.ds(k * LANES, LANES)
            gxv = gx[v16]
            gyv = gy[v16]
            rxv = rx[v16]
            ryv = ry[v16]
            v = vj[v16]
            r2 = (rxv * rxv + ryv * ryv) + EPS
            qq = cf[v16] / r2
            cd[v16] = ((qq * rxv - rgxj[v16] * v) * gxv
                       + (qq * ryv - rgyj[v16] * v) * gyv)
            cgx[v16] = gxv * v
            cgy[v16] = gyv * v

        _wait_all([pltpu.async_copy(cd, ad.at[ii], sem, add=True),
                   pltpu.async_copy(cgx, agx.at[ii], sem, add=True),
                   pltpu.async_copy(cgy, agy.at[ii], sem, add=True)])

    plsc.subcore_barrier()
    _dump_accs(zbuf, accs, (od_h, ogx_h, ogy_h),
               tile_base, c * NP_PAD + tile_base)


def _pinv_rg_body(n00a, n00b, n01a, n01b, n10a, n10b, n11a, n11b,
                  sxa, sxb, sya, syb, rgx, rgy):
    m00 = n00a[...] + n00b[...]
    m01 = n01a[...] + n01b[...]
    m10 = n10a[...] + n10b[...]
    m11 = n11a[...] + n11b[...]
    sx = sxa[...] + sxb[...]
    sy = sya[...] + syb[...]
    det = m00 * m11 - m01 * m10
    dets = jnp.where(jnp.abs(det) > 1e-7, det,
                     jnp.where(det >= 0, 1e-7, -1e-7))
    l00 = m11 / dets
    l01 = -m01 / dets
    l10 = -m10 / dets
    l11 = m00 / dets
    rgx[...] = -(l00 * sx + l01 * sy)
    rgy[...] = -(l10 * sx + l11 * sy)


def _final_body(da, db, gxa, gxb, gya, gyb, rgx, rgy, o):
    gvx = gxa[...] + gxb[...]
    gvy = gya[...] + gyb[...]
    o[...] = ((da[...] + db[...])
              - (rgx[...] * gvx + rgy[...] * gvy)) * OUT_SCALE


def _tc_call(body, n_in, n_out, args):
    bs = pl.BlockSpec((8, 128), lambda g: (g, 0))
    shape = jax.ShapeDtypeStruct((ROWS, 128), jnp.float32)
    f = pl.pallas_call(
        body,
        grid=(GRID,),
        in_specs=[bs] * n_in,
        out_specs=[bs] * n_out if n_out > 1 else bs,
        out_shape=[shape] * n_out if n_out > 1 else shape,
    )
    return f(*args)


def _split2(flat):
    r = flat.reshape(2, ROWS, 128)
    return r[0], r[1]


def kernel(positions, volumes, distances, radialDistances, density, i, j):
    ii = i.astype(jnp.int32)
    jj = j.astype(jnp.int32)
    pad = NP_PAD - N
    px = jnp.pad(positions[:, 0], (0, pad))
    py = jnp.pad(positions[:, 1], (0, pad))
    vol = jnp.pad(volumes, (0, pad))
    den = jnp.pad(density, (0, pad))
    dx = distances[:, 0]
    dy = distances[:, 1]
    rad = radialDistances

    mesh = _mesh()
    ebuf = lambda: pltpu.VMEM((B,), jnp.float32)
    eibuf = lambda: pltpu.VMEM((B,), jnp.int32)
    zbuf = pltpu.VMEM((PER_TILE,), jnp.float32)
    shr = lambda: pltpu.VMEM_SHARED((NP_PAD,), jnp.float32)
    oshape = jax.ShapeDtypeStruct((2 * NP_PAD,), jnp.float32)
    eshape = jax.ShapeDtypeStruct((E,), jnp.float32)
    cparams = pltpu.CompilerParams(use_tc_tiling_on_sc=False)

    sweep1 = pl.kernel(
        _sweep_one,
        out_type=(oshape,) * 6 + (eshape,) * 6,
        mesh=mesh,
        compiler_params=cparams,
        scratch_types=(
            [eibuf(), eibuf()] + [ebuf()] * 21
            + [zbuf] + [shr() for _ in range(10)]
            + [pltpu.SemaphoreType.DMA, pltpu.SemaphoreType.DMA]
        ),
    )
    (nm00, nm01, nm10, nm11, sxp, syp,
     egx, egy, erx, ery, ecf, evj) = sweep1(
        ii, jj, dx, dy, rad, px, py, vol, den)

    parts = (_split2(nm00) + _split2(nm01) + _split2(nm10) + _split2(nm11)
             + _split2(sxp) + _split2(syp))
    rgx, rgy = _tc_call(_pinv_rg_body, 12, 2, parts)
    rgx_f = rgx.reshape(NP_PAD)
    rgy_f = rgy.reshape(NP_PAD)

    sweep2 = pl.kernel(
        _sweep_two,
        out_type=(oshape,) * 3,
        mesh=mesh,
        compiler_params=cparams,
        scratch_types=(
            [eibuf(), eibuf()] + [ebuf()] * 11
            + [zbuf] + [shr() for _ in range(5)] + [pltpu.SemaphoreType.DMA]
        ),
    )
    dd2, gvx2, gvy2 = sweep2(ii, jj, egx, egy, erx, ery, ecf, evj,
                             rgx_f, rgy_f)

    out = _tc_call(_final_body, 8, 1,
                   _split2(dd2) + _split2(gvx2) + _split2(gvy2)
                   + (rgx, rgy)).reshape(NP_PAD)
    return out[:N]
